# baseline (speedup 1.0000x reference)
import jax
import jax.numpy as jnp
from jax.experimental import pallas as pl
from jax.experimental.pallas import tpu as pltpu


def _make_body(d, h, w):
    def _body(ct_ref, rt_ref, out_ref, plane_ref):
        @pl.when(pl.program_id(0) == 0)
        def _():
            ct = ct_ref[...]
            rt = rt_ref[...]
            col = jnp.broadcast_to(ct[:, None, :], (d, h, w)).reshape(d, h * w)
            row = jnp.broadcast_to(rt[:, :, None], (d, h, w)).reshape(d, h * w)
            plane_ref[:d, :] = col
            plane_ref[d:, :] = row

        out_ref[...] = plane_ref[...]

    return _body


def kernel(x, row_embed, col_embed):
    b = x.shape[0]
    h, w = x.shape[-2], x.shape[-1]
    d = row_embed.shape[1]
    ceT = col_embed[:w].T
    reT = row_embed[:h].T
    out = pl.pallas_call(
        _make_body(d, h, w),
        grid=(b,),
        in_specs=[
            pl.BlockSpec((d, w), lambda i: (0, 0)),
            pl.BlockSpec((d, h), lambda i: (0, 0)),
        ],
        out_specs=pl.BlockSpec((2 * d, h * w), lambda i: (i, 0)),
        out_shape=jax.ShapeDtypeStruct((b * 2 * d, h * w), jnp.float32),
        scratch_shapes=[pltpu.VMEM((2 * d, h * w), jnp.float32)],
    )(ceT, reT)
    return out.reshape(b, 2 * d, h, w)


# speedup vs baseline: 1.0629x; 1.0629x over previous
import jax
import jax.numpy as jnp
from jax import lax
from jax.experimental import pallas as pl
from jax.experimental.pallas import tpu as pltpu


def _make_body(d, h, w):
    hw = h * w

    def _body(re_ref, ce_ref, out_ref, plane_ref):
        @pl.when(pl.program_id(0) == 0)
        def _():
            ce = ce_ref[0:w, :]
            re = re_ref[0:h, :]
            j = lax.broadcasted_iota(jnp.int32, (w, hw), 0)
            p = lax.broadcasted_iota(jnp.int32, (w, hw), 1)
            tile_pat = (p % w == j).astype(jnp.float32)
            rep_pat = (p // w == j).astype(jnp.float32)
            dn = (((0,), (0,)), ((), ()))
            plane_ref[:d, :] = lax.dot_general(
                ce, tile_pat, dn, preferred_element_type=jnp.float32)
            plane_ref[d:, :] = lax.dot_general(
                re, rep_pat, dn, preferred_element_type=jnp.float32)

        out_ref[...] = plane_ref[...]

    return _body


def kernel(x, row_embed, col_embed):
    b = x.shape[0]
    h, w = x.shape[-2], x.shape[-1]
    n, d = row_embed.shape
    out = pl.pallas_call(
        _make_body(d, h, w),
        grid=(b,),
        in_specs=[
            pl.BlockSpec((n, d), lambda i: (0, 0)),
            pl.BlockSpec((n, d), lambda i: (0, 0)),
        ],
        out_specs=pl.BlockSpec((2 * d, h * w), lambda i: (i, 0)),
        out_shape=jax.ShapeDtypeStruct((b * 2 * d, h * w), jnp.float32),
        scratch_shapes=[pltpu.VMEM((2 * d, h * w), jnp.float32)],
    )(row_embed, col_embed)
    return out.reshape(b, 2 * d, h, w)


# speedup vs baseline: 9.6527x; 9.0812x over previous
import jax
import jax.numpy as jnp
from jax import lax
from jax.experimental import pallas as pl
from jax.experimental.pallas import tpu as pltpu


def _make_body(d, h, w):
    hw = h * w

    def _body(re_ref, ce_ref, out_ref, plane_ref):
        @pl.when(pl.program_id(0) == 0)
        def _():
            ce = ce_ref[0:w, :]
            re = re_ref[0:h, :]
            r = lax.broadcasted_iota(jnp.int32, (hw, w), 0)
            k = lax.broadcasted_iota(jnp.int32, (hw, w), 1)
            sel_j = (r % w == k).astype(jnp.float32)
            sel_i = (r // w == k).astype(jnp.float32)
            plane_ref[:, :d] = jnp.dot(
                sel_j, ce, preferred_element_type=jnp.float32)
            plane_ref[:, d:] = jnp.dot(
                sel_i, re, preferred_element_type=jnp.float32)

        out_ref[...] = plane_ref[...]

    return _body


def kernel(x, row_embed, col_embed):
    b = x.shape[0]
    h, w = x.shape[-2], x.shape[-1]
    n, d = row_embed.shape
    out = pl.pallas_call(
        _make_body(d, h, w),
        grid=(b,),
        in_specs=[
            pl.BlockSpec((n, d), lambda i: (0, 0)),
            pl.BlockSpec((n, d), lambda i: (0, 0)),
        ],
        out_specs=pl.BlockSpec((h * w, 2 * d), lambda i: (i, 0)),
        out_shape=jax.ShapeDtypeStruct((b * h * w, 2 * d), jnp.float32),
        scratch_shapes=[pltpu.VMEM((h * w, 2 * d), jnp.float32)],
    )(row_embed, col_embed)
    return out.reshape(b, h, w, 2 * d).transpose(0, 3, 1, 2)


# speedup vs baseline: 10.9350x; 1.1328x over previous
import jax
import jax.numpy as jnp
from jax import lax
from jax.experimental import pallas as pl
from jax.experimental.pallas import tpu as pltpu


def _make_body(b, d, h, w):
    hw = h * w
    half = hw // 2

    def _body(re_ref, ce_ref, out_ref, plane_ref, sem):
        ce = ce_ref[0:w, :]
        re = re_ref[0:h, :]
        copies = []
        for chunk in range(2):
            r0 = chunk * half
            r = r0 + lax.broadcasted_iota(jnp.int32, (half, w), 0)
            k = lax.broadcasted_iota(jnp.int32, (half, w), 1)
            sel_j = (r % w == k).astype(jnp.float32)
            sel_i = (r // w == k).astype(jnp.float32)
            plane_ref[pl.ds(r0, half), :d] = jnp.dot(
                sel_j, ce, preferred_element_type=jnp.float32)
            plane_ref[pl.ds(r0, half), d:] = jnp.dot(
                sel_i, re, preferred_element_type=jnp.float32)
            for bi in range(b):
                cp = pltpu.make_async_copy(
                    plane_ref.at[pl.ds(r0, half), :],
                    out_ref.at[pl.ds(bi * hw + r0, half), :],
                    sem)
                cp.start()
                copies.append(cp)
        for cp in copies:
            cp.wait()

    return _body


def kernel(x, row_embed, col_embed):
    b = x.shape[0]
    h, w = x.shape[-2], x.shape[-1]
    n, d = row_embed.shape
    out = pl.pallas_call(
        _make_body(b, d, h, w),
        in_specs=[
            pl.BlockSpec(memory_space=pltpu.VMEM),
            pl.BlockSpec(memory_space=pltpu.VMEM),
        ],
        out_specs=pl.BlockSpec(memory_space=pl.ANY),
        out_shape=jax.ShapeDtypeStruct((b * h * w, 2 * d), jnp.float32),
        scratch_shapes=[
            pltpu.VMEM((h * w, 2 * d), jnp.float32),
            pltpu.SemaphoreType.DMA,
        ],
    )(row_embed, col_embed)
    return out.reshape(b, h, w, 2 * d).transpose(0, 3, 1, 2)


# speedup vs baseline: 11.4992x; 1.0516x over previous
import jax
import jax.numpy as jnp
from jax.experimental import pallas as pl
from jax.experimental.pallas import tpu as pltpu


def _make_body(b, d, h, w, nchunk):
    hw = h * w
    gpc = h // nchunk

    def _body(re_ref, ce_ref, out_ref, plane_ref, sem):
        ce = ce_ref[0:w, :]
        copies = []
        for chunk in range(nchunk):
            for g in range(gpc):
                i = chunk * gpc + g
                plane_ref[pl.ds(i * w, w), :d] = ce
                plane_ref[pl.ds(i * w, w), d:] = jnp.broadcast_to(
                    re_ref[i:i + 1, :], (w, d))
            r0 = chunk * gpc * w
            nrows = gpc * w
            for bi in range(b):
                cp = pltpu.make_async_copy(
                    plane_ref.at[pl.ds(r0, nrows), :],
                    out_ref.at[pl.ds(bi * hw + r0, nrows), :],
                    sem)
                cp.start()
                copies.append(cp)
        for cp in copies:
            cp.wait()

    return _body


def kernel(x, row_embed, col_embed):
    b = x.shape[0]
    h, w = x.shape[-2], x.shape[-1]
    n, d = row_embed.shape
    out = pl.pallas_call(
        _make_body(b, d, h, w, nchunk=4),
        in_specs=[
            pl.BlockSpec(memory_space=pltpu.VMEM),
            pl.BlockSpec(memory_space=pltpu.VMEM),
        ],
        out_specs=pl.BlockSpec(memory_space=pl.ANY),
        out_shape=jax.ShapeDtypeStruct((b * h * w, 2 * d), jnp.float32),
        scratch_shapes=[
            pltpu.VMEM((h * w, 2 * d), jnp.float32),
            pltpu.SemaphoreType.DMA,
        ],
    )(row_embed, col_embed)
    return out.reshape(b, h, w, 2 * d).transpose(0, 3, 1, 2)
